# Initial kernel scaffold; baseline (speedup 1.0000x reference)
import jax
import jax.numpy as jnp
from jax.experimental import pallas as pl


def kernel(tok_emb, pos_emb, seg_emb, emb_gamma, emb_beta, ib_emb_gate, wqkv, bqkv, wd, w1, w2, vecs_d, vecs_h, token_ids, seg_ids, mask):
    raise NotImplementedError("write your pallas kernel here")


# speedup vs baseline: 1.2632x; 1.2632x over previous
import math
import functools

import jax
import jax.numpy as jnp
from jax import lax
from jax.experimental import pallas as pl
from jax.experimental.pallas import tpu as pltpu

EPS = 1e-12
VMEM_LIMIT_BYTES = 48 * 1024 * 1024

_VD_GATE_ATTN, _VD_BD, _VD_G1, _VD_BE1, _VD_GATE_IB1, _VD_B2F, _VD_G2, _VD_BE2, _VD_GATE_OUT = range(9)
_VH_B1F, _VH_GATE_IB2 = range(2)


def _layernorm(x, g, b):
    mean = jnp.mean(x, axis=-1, keepdims=True)
    var = jnp.mean((x - mean) ** 2, axis=-1, keepdims=True)
    return g * ((x - mean) * lax.rsqrt(var + EPS)) + b


def _gelu(x):
    return x * 0.5 * (1.0 + lax.erf(x / math.sqrt(2.0)))


def _encoder_layer_kernel(x_ref, neg_ref, wqkv_ref, bqkv_ref, wd_ref,
                          w1_ref, w2_ref, vd_ref, vh_ref, out_ref,
                          *, n_heads):
    f32 = jnp.float32
    bf16 = jnp.bfloat16
    BB, S, D = x_ref.shape
    Dh = D // n_heads

    x = x_ref[...].reshape(BB * S, D)
    qkv = jnp.dot(x, wqkv_ref[...],
                  preferred_element_type=f32) + bqkv_ref[...]
    q = qkv[:, :D].astype(bf16)
    k = qkv[:, D:2 * D].astype(bf16)
    v = qkv[:, 2 * D:].astype(bf16)

    vd = vd_ref[...]
    vh = vh_ref[...]
    gate_attn = vd[_VD_GATE_ATTN:_VD_GATE_ATTN + 1, :]

    row_ctx = []
    for r in range(BB):
        rs = slice(r * S, (r + 1) * S)
        neg = neg_ref[r]
        head_ctx = []
        for h in range(n_heads):
            sl = slice(h * Dh, (h + 1) * Dh)
            s = lax.dot_general(q[rs, sl], k[rs, sl], (((1,), (1,)), ((), ())),
                                preferred_element_type=f32)
            s = s - neg
            s = s - jnp.max(s, axis=-1, keepdims=True)
            p = jnp.exp(s)
            p = p * pl.reciprocal(jnp.sum(p, axis=-1, keepdims=True), approx=True)
            head_ctx.append(jnp.dot(p.astype(bf16), v[rs, sl],
                                    preferred_element_type=f32))
        row_ctx.append(jnp.concatenate(head_ctx, axis=1))
    ctx = jnp.concatenate(row_ctx, axis=0) * gate_attn

    attn = jnp.dot(ctx.astype(bf16), wd_ref[...],
                   preferred_element_type=f32) + vd[_VD_BD:_VD_BD + 1, :]
    x1 = _layernorm(x.astype(f32) + attn,
                    vd[_VD_G1:_VD_G1 + 1, :], vd[_VD_BE1:_VD_BE1 + 1, :])

    y = x1 * vd[_VD_GATE_IB1:_VD_GATE_IB1 + 1, :]
    y = jnp.dot(y.astype(bf16), w1_ref[...],
                preferred_element_type=f32) + vh[_VH_B1F:_VH_B1F + 1, :]
    y = _gelu(y)
    y = y * vh[_VH_GATE_IB2:_VH_GATE_IB2 + 1, :]
    y = jnp.dot(y.astype(bf16), w2_ref[...],
                preferred_element_type=f32) + vd[_VD_B2F:_VD_B2F + 1, :]
    x2 = _layernorm(x1 + y, vd[_VD_G2:_VD_G2 + 1, :], vd[_VD_BE2:_VD_BE2 + 1, :])

    out = x2 * vd[_VD_GATE_OUT:_VD_GATE_OUT + 1, :]
    out_ref[...] = out.astype(out_ref.dtype).reshape(BB, S, D)


def _encoder_layer(x, neg3, wqkv_l, bqkv_l, wd_l, w1_l, w2_l, vd_l, vh_l,
                   n_heads, bb):
    B, S, D = x.shape
    Hid = w1_l.shape[1]
    body = functools.partial(_encoder_layer_kernel, n_heads=n_heads)
    return pl.pallas_call(
        body,
        out_shape=jax.ShapeDtypeStruct((B, S, D), jnp.bfloat16),
        grid=(B // bb,),
        in_specs=[
            pl.BlockSpec((bb, S, D), lambda b: (b, 0, 0)),
            pl.BlockSpec((bb, 1, S), lambda b: (b, 0, 0)),
            pl.BlockSpec((D, 3 * D), lambda b: (0, 0)),
            pl.BlockSpec((1, 3 * D), lambda b: (0, 0)),
            pl.BlockSpec((D, D), lambda b: (0, 0)),
            pl.BlockSpec((D, Hid), lambda b: (0, 0)),
            pl.BlockSpec((Hid, D), lambda b: (0, 0)),
            pl.BlockSpec((9, D), lambda b: (0, 0)),
            pl.BlockSpec((2, Hid), lambda b: (0, 0)),
        ],
        out_specs=pl.BlockSpec((bb, S, D), lambda b: (b, 0, 0)),
        compiler_params=pltpu.CompilerParams(
            dimension_semantics=("parallel",),
            vmem_limit_bytes=VMEM_LIMIT_BYTES),
    )(x, neg3, wqkv_l, bqkv_l, wd_l, w1_l, w2_l, vd_l, vh_l)


def kernel(tok_emb, pos_emb, seg_emb, emb_gamma, emb_beta, ib_emb_gate,
           wqkv, bqkv, wd, w1, w2, vecs_d, vecs_h,
           token_ids, seg_ids, mask):
    L = wqkv.shape[0]
    B, S = token_ids.shape
    n_heads = 12
    bb = 4

    tok = jnp.take(tok_emb, token_ids, axis=0)
    pos = pos_emb[:S][None, :, :]
    seg = jnp.take(seg_emb, seg_ids, axis=0)
    emb = (tok + pos + seg).astype(jnp.float32)
    x = _layernorm(emb, emb_gamma, emb_beta)
    x = (x * ib_emb_gate).astype(jnp.bfloat16)

    neg3 = (10000.0 * (1.0 - mask.astype(jnp.float32)))[:, None, :]

    for l in range(L):
        x = _encoder_layer(x, neg3, wqkv[l], bqkv[l], wd[l], w1[l], w2[l],
                           vecs_d[l], vecs_h[l], n_heads, bb)
    return x


# speedup vs baseline: 2.1195x; 1.6778x over previous
import math
import functools

import jax
import jax.numpy as jnp
from jax import lax
from jax.experimental import pallas as pl
from jax.experimental.pallas import tpu as pltpu

EPS = 1e-12
VMEM_LIMIT_BYTES = 48 * 1024 * 1024

_VD_GATE_ATTN, _VD_BD, _VD_G1, _VD_BE1, _VD_GATE_IB1, _VD_B2F, _VD_G2, _VD_BE2, _VD_GATE_OUT = range(9)
_VH_B1F, _VH_GATE_IB2 = range(2)


def _layernorm(x, g, b):
    mean = jnp.mean(x, axis=-1, keepdims=True)
    var = jnp.mean((x - mean) ** 2, axis=-1, keepdims=True)
    return g * ((x - mean) * lax.rsqrt(var + EPS)) + b


def _gelu(x):
    return x * 0.5 * (1.0 + lax.erf(x / math.sqrt(2.0)))


def _encoder_layer_kernel(x_ref, neg_ref, wqkv_ref, bqkv_ref, wd_ref,
                          w1_ref, w2_ref, vd_ref, vh_ref, out_ref,
                          s_ref, p_ref, g_ref, *, n_heads, hid_chunks):
    f32 = jnp.float32
    bf16 = jnp.bfloat16
    BB, S, D = x_ref.shape
    H = n_heads
    Dh = D // H
    M = BB * S

    x = x_ref[...].reshape(M, D)
    qkv = jnp.dot(x, wqkv_ref[...],
                  preferred_element_type=f32) + bqkv_ref[...]
    q = qkv[:, :D].astype(bf16)
    k = qkv[:, D:2 * D].astype(bf16)
    v = qkv[:, 2 * D:].astype(bf16)

    vd = vd_ref[...]
    vh = vh_ref[...]

    for r in range(BB):
        rs = slice(r * S, (r + 1) * S)
        neg = neg_ref[r]
        base = r * H * S
        for h in range(H):
            sl = slice(h * Dh, (h + 1) * Dh)
            s = lax.dot_general(q[rs, sl], k[rs, sl], (((1,), (1,)), ((), ())),
                                preferred_element_type=f32)
            s_ref[base + h * S:base + (h + 1) * S, :] = s - neg
        srow = s_ref[base:base + H * S, :]
        srow = srow - jnp.max(srow, axis=-1, keepdims=True)
        e = jnp.exp(srow)
        p = e * pl.reciprocal(jnp.sum(e, axis=-1, keepdims=True), approx=True)
        p_ref[base:base + H * S, :] = p.astype(bf16)

    row_ctx = []
    for r in range(BB):
        rs = slice(r * S, (r + 1) * S)
        base = r * H * S
        head_ctx = []
        for h in range(H):
            head_ctx.append(jnp.dot(p_ref[base + h * S:base + (h + 1) * S, :],
                                    v[rs, h * Dh:(h + 1) * Dh],
                                    preferred_element_type=f32))
        row_ctx.append(jnp.concatenate(head_ctx, axis=1))
    ctx = jnp.concatenate(row_ctx, axis=0) * vd[_VD_GATE_ATTN:_VD_GATE_ATTN + 1, :]

    attn = jnp.dot(ctx.astype(bf16), wd_ref[...],
                   preferred_element_type=f32) + vd[_VD_BD:_VD_BD + 1, :]
    x1 = _layernorm(x.astype(f32) + attn,
                    vd[_VD_G1:_VD_G1 + 1, :], vd[_VD_BE1:_VD_BE1 + 1, :])

    y0 = (x1 * vd[_VD_GATE_IB1:_VD_GATE_IB1 + 1, :]).astype(bf16)
    Hid = w1_ref.shape[1]
    CH = Hid // hid_chunks
    for c in range(hid_chunks):
        cs = slice(c * CH, (c + 1) * CH)
        t = jnp.dot(y0, w1_ref[:, cs],
                    preferred_element_type=f32) + vh[_VH_B1F:_VH_B1F + 1, cs]
        g_ref[:, cs] = (_gelu(t) * vh[_VH_GATE_IB2:_VH_GATE_IB2 + 1, cs]).astype(bf16)
    y = jnp.dot(g_ref[...], w2_ref[...],
                preferred_element_type=f32) + vd[_VD_B2F:_VD_B2F + 1, :]
    x2 = _layernorm(x1 + y, vd[_VD_G2:_VD_G2 + 1, :], vd[_VD_BE2:_VD_BE2 + 1, :])

    out = x2 * vd[_VD_GATE_OUT:_VD_GATE_OUT + 1, :]
    out_ref[...] = out.astype(out_ref.dtype).reshape(BB, S, D)


def _encoder_layer(x, neg3, wqkv_l, bqkv_l, wd_l, w1_l, w2_l, vd_l, vh_l,
                   n_heads, bb):
    B, S, D = x.shape
    Hid = w1_l.shape[1]
    body = functools.partial(_encoder_layer_kernel, n_heads=n_heads,
                             hid_chunks=4)
    return pl.pallas_call(
        body,
        out_shape=jax.ShapeDtypeStruct((B, S, D), jnp.bfloat16),
        grid=(B // bb,),
        in_specs=[
            pl.BlockSpec((bb, S, D), lambda b: (b, 0, 0)),
            pl.BlockSpec((bb, 1, S), lambda b: (b, 0, 0)),
            pl.BlockSpec((D, 3 * D), lambda b: (0, 0)),
            pl.BlockSpec((1, 3 * D), lambda b: (0, 0)),
            pl.BlockSpec((D, D), lambda b: (0, 0)),
            pl.BlockSpec((D, Hid), lambda b: (0, 0)),
            pl.BlockSpec((Hid, D), lambda b: (0, 0)),
            pl.BlockSpec((9, D), lambda b: (0, 0)),
            pl.BlockSpec((2, Hid), lambda b: (0, 0)),
        ],
        out_specs=pl.BlockSpec((bb, S, D), lambda b: (b, 0, 0)),
        scratch_shapes=[
            pltpu.VMEM((bb * n_heads * S, S), jnp.float32),
            pltpu.VMEM((bb * n_heads * S, S), jnp.bfloat16),
            pltpu.VMEM((bb * S, Hid), jnp.bfloat16),
        ],
        compiler_params=pltpu.CompilerParams(
            dimension_semantics=("parallel",),
            vmem_limit_bytes=VMEM_LIMIT_BYTES),
    )(x, neg3, wqkv_l, bqkv_l, wd_l, w1_l, w2_l, vd_l, vh_l)


def kernel(tok_emb, pos_emb, seg_emb, emb_gamma, emb_beta, ib_emb_gate,
           wqkv, bqkv, wd, w1, w2, vecs_d, vecs_h,
           token_ids, seg_ids, mask):
    L = wqkv.shape[0]
    B, S = token_ids.shape
    n_heads = 12
    bb = 4

    tok = jnp.take(tok_emb, token_ids, axis=0)
    pos = pos_emb[:S][None, :, :]
    seg = jnp.take(seg_emb, seg_ids, axis=0)
    emb = (tok + pos + seg).astype(jnp.float32)
    x = _layernorm(emb, emb_gamma, emb_beta)
    x = (x * ib_emb_gate).astype(jnp.bfloat16)

    neg3 = (10000.0 * (1.0 - mask.astype(jnp.float32)))[:, None, :]

    for l in range(L):
        x = _encoder_layer(x, neg3, wqkv[l], bqkv[l], wd[l], w1[l], w2[l],
                           vecs_d[l], vecs_h[l], n_heads, bb)
    return x


# speedup vs baseline: 2.1282x; 1.0041x over previous
import math
import functools

import jax
import jax.numpy as jnp
from jax import lax
from jax.experimental import pallas as pl
from jax.experimental.pallas import tpu as pltpu

EPS = 1e-12
VMEM_LIMIT_BYTES = 56 * 1024 * 1024

_VD_GATE_ATTN, _VD_BD, _VD_G1, _VD_BE1, _VD_GATE_IB1, _VD_B2F, _VD_G2, _VD_BE2, _VD_GATE_OUT = range(9)
_VH_B1F, _VH_GATE_IB2 = range(2)


def _layernorm(x, g, b):
    mean = jnp.mean(x, axis=-1, keepdims=True)
    var = jnp.mean((x - mean) ** 2, axis=-1, keepdims=True)
    return g * ((x - mean) * lax.rsqrt(var + EPS)) + b


def _gelu(x):
    return x * 0.5 * (1.0 + lax.erf(x / math.sqrt(2.0)))


def _encoder_layer_kernel(x_ref, neg_ref, wqkv_ref, bqkv_ref, wd_ref,
                          w1_ref, w2_ref, vd_ref, vh_ref, out_ref,
                          s_ref, p_ref, g_ref, *, n_heads, hid_chunks):
    f32 = jnp.float32
    bf16 = jnp.bfloat16
    BB, S, D = x_ref.shape
    H = n_heads
    Dh = D // H
    M = BB * S

    x = x_ref[...].reshape(M, D)
    qkv = jnp.dot(x, wqkv_ref[...],
                  preferred_element_type=f32) + bqkv_ref[...]
    q = qkv[:, :D].astype(bf16)
    k = qkv[:, D:2 * D].astype(bf16)
    v = qkv[:, 2 * D:].astype(bf16)

    vd = vd_ref[...]
    vh = vh_ref[...]

    for r in range(BB):
        rs = slice(r * S, (r + 1) * S)
        neg = neg_ref[r]
        base = r * H * S
        for h in range(H):
            sl = slice(h * Dh, (h + 1) * Dh)
            s = lax.dot_general(q[rs, sl], k[rs, sl], (((1,), (1,)), ((), ())),
                                preferred_element_type=f32)
            s_ref[base + h * S:base + (h + 1) * S, :] = s - neg
        srow = s_ref[base:base + H * S, :]
        srow = srow - jnp.max(srow, axis=-1, keepdims=True)
        e = jnp.exp(srow)
        p = e * pl.reciprocal(jnp.sum(e, axis=-1, keepdims=True), approx=True)
        p_ref[base:base + H * S, :] = p.astype(bf16)

    row_ctx = []
    for r in range(BB):
        rs = slice(r * S, (r + 1) * S)
        base = r * H * S
        head_ctx = []
        for h in range(H):
            head_ctx.append(jnp.dot(p_ref[base + h * S:base + (h + 1) * S, :],
                                    v[rs, h * Dh:(h + 1) * Dh],
                                    preferred_element_type=f32))
        row_ctx.append(jnp.concatenate(head_ctx, axis=1))
    ctx = jnp.concatenate(row_ctx, axis=0) * vd[_VD_GATE_ATTN:_VD_GATE_ATTN + 1, :]

    attn = jnp.dot(ctx.astype(bf16), wd_ref[...],
                   preferred_element_type=f32) + vd[_VD_BD:_VD_BD + 1, :]
    x1 = _layernorm(x.astype(f32) + attn,
                    vd[_VD_G1:_VD_G1 + 1, :], vd[_VD_BE1:_VD_BE1 + 1, :])

    y0 = (x1 * vd[_VD_GATE_IB1:_VD_GATE_IB1 + 1, :]).astype(bf16)
    Hid = w1_ref.shape[1]
    CH = Hid // hid_chunks
    for c in range(hid_chunks):
        cs = slice(c * CH, (c + 1) * CH)
        t = jnp.dot(y0, w1_ref[:, cs],
                    preferred_element_type=f32) + vh[_VH_B1F:_VH_B1F + 1, cs]
        g_ref[:, cs] = (_gelu(t) * vh[_VH_GATE_IB2:_VH_GATE_IB2 + 1, cs]).astype(bf16)
    y = jnp.dot(g_ref[...], w2_ref[...],
                preferred_element_type=f32) + vd[_VD_B2F:_VD_B2F + 1, :]
    x2 = _layernorm(x1 + y, vd[_VD_G2:_VD_G2 + 1, :], vd[_VD_BE2:_VD_BE2 + 1, :])

    out = x2 * vd[_VD_GATE_OUT:_VD_GATE_OUT + 1, :]
    out_ref[...] = out.astype(out_ref.dtype).reshape(BB, S, D)


def _encoder_layer(x, neg3, wqkv_l, bqkv_l, wd_l, w1_l, w2_l, vd_l, vh_l,
                   n_heads, bb):
    B, S, D = x.shape
    Hid = w1_l.shape[1]
    body = functools.partial(_encoder_layer_kernel, n_heads=n_heads,
                             hid_chunks=4)
    return pl.pallas_call(
        body,
        out_shape=jax.ShapeDtypeStruct((B, S, D), jnp.bfloat16),
        grid=(B // bb,),
        in_specs=[
            pl.BlockSpec((bb, S, D), lambda b: (b, 0, 0)),
            pl.BlockSpec((bb, 1, S), lambda b: (b, 0, 0)),
            pl.BlockSpec((D, 3 * D), lambda b: (0, 0)),
            pl.BlockSpec((1, 3 * D), lambda b: (0, 0)),
            pl.BlockSpec((D, D), lambda b: (0, 0)),
            pl.BlockSpec((D, Hid), lambda b: (0, 0)),
            pl.BlockSpec((Hid, D), lambda b: (0, 0)),
            pl.BlockSpec((9, D), lambda b: (0, 0)),
            pl.BlockSpec((2, Hid), lambda b: (0, 0)),
        ],
        out_specs=pl.BlockSpec((bb, S, D), lambda b: (b, 0, 0)),
        scratch_shapes=[
            pltpu.VMEM((bb * n_heads * S, S), jnp.float32),
            pltpu.VMEM((bb * n_heads * S, S), jnp.bfloat16),
            pltpu.VMEM((bb * S, Hid), jnp.bfloat16),
        ],
        compiler_params=pltpu.CompilerParams(
            dimension_semantics=("parallel",),
            vmem_limit_bytes=VMEM_LIMIT_BYTES),
    )(x, neg3, wqkv_l, bqkv_l, wd_l, w1_l, w2_l, vd_l, vh_l)


def kernel(tok_emb, pos_emb, seg_emb, emb_gamma, emb_beta, ib_emb_gate,
           wqkv, bqkv, wd, w1, w2, vecs_d, vecs_h,
           token_ids, seg_ids, mask):
    L = wqkv.shape[0]
    B, S = token_ids.shape
    n_heads = 12
    bb = 8

    tok = jnp.take(tok_emb, token_ids, axis=0)
    pos = pos_emb[:S][None, :, :]
    seg = jnp.take(seg_emb, seg_ids, axis=0)
    emb = (tok + pos + seg).astype(jnp.float32)
    x = _layernorm(emb, emb_gamma, emb_beta)
    x = (x * ib_emb_gate).astype(jnp.bfloat16)

    neg3 = (10000.0 * (1.0 - mask.astype(jnp.float32)))[:, None, :]

    for l in range(L):
        x = _encoder_layer(x, neg3, wqkv[l], bqkv[l], wd[l], w1[l], w2[l],
                           vecs_d[l], vecs_h[l], n_heads, bb)
    return x
